# Initial kernel scaffold; baseline (speedup 1.0000x reference)
import jax
import jax.numpy as jnp
from jax.experimental import pallas as pl


def kernel(x, W1, b1, W2, b2):
    raise NotImplementedError("write your pallas kernel here")


# speedup vs baseline: 1.4945x; 1.4945x over previous
import jax
import jax.numpy as jnp
import numpy as np
from jax.experimental import pallas as pl
from jax.experimental.pallas import tpu as pltpu

_C = 1024
_T = 576
_HEADS = 16
_HD = _C // _HEADS
_SZ = 128
_CA = _C + _SZ
_HIGHEST = jax.lax.Precision.HIGHEST


def _split_even_odd(v):
    p, c = v.shape
    r = v.reshape(p // 2, 2, c)
    return r[:, 0, :], r[:, 1, :]


def _tile8(s):
    return jnp.concatenate([s] * (_C // _SZ), axis=1)


def _metric(x):
    m = x[:, 0:_HD]
    for h in range(1, _HEADS):
        m = m + x[:, h * _HD : (h + 1) * _HD]
    m = m / np.float32(_HEADS)
    m2 = m * m
    p8 = m2[:, 0:8]
    for k in range(1, 8):
        p8 = p8 + m2[:, 8 * k : 8 * k + 8]
    q = p8[:, 0:4] + p8[:, 4:8]
    rr = q[:, 0:2] + q[:, 2:4]
    s = rr[:, 0:1] + rr[:, 1:2]
    return m / jnp.sqrt(s)


def _row_of_col(v, n):
    eye = (
        jax.lax.broadcasted_iota(jnp.int32, (n, n), 0)
        == jax.lax.broadcasted_iota(jnp.int32, (n, n), 1)
    ).astype(jnp.float32)
    return jax.lax.dot_general(
        v, eye, (((0,), (0,)), ((), ())), precision=_HIGHEST
    )


def _merge_step(D, acc, half, r):
    x = D[:, :_C]
    s = D[:, _C:]
    metric = _metric(x)
    a, b = _split_even_odd(metric)
    scores = jax.lax.dot_general(a, b, (((1,), (1,)), ((), ())))
    nm = jnp.max(scores, axis=1, keepdims=True)
    jj = jax.lax.broadcasted_iota(jnp.int32, (half, half), 1)
    ii = jax.lax.broadcasted_iota(jnp.int32, (half, half), 0)
    eq = scores == nm
    jsel = jnp.min(jnp.where(eq, jj, half), axis=1, keepdims=True)
    n2t = jj == jsel
    nm_t = _row_of_col(nm, half)
    before = (nm_t > nm) | ((nm_t == nm) & (jj < ii))
    rank = jnp.sum(before.astype(jnp.float32), axis=1, keepdims=True)
    rank_t = _row_of_col(rank, half)
    n2f = n2t.astype(jnp.float32)
    samedst = jax.lax.dot_general(n2f, n2f, (((1,), (1,)), ((), ())))
    earlier = (rank_t < rank).astype(jnp.float32)
    occ = jnp.sum(samedst * earlier, axis=1, keepdims=True)

    t_aug = jnp.concatenate([x * _tile8(s), s], axis=1)
    te, to = _split_even_odd(t_aug)
    is_src = rank < r
    occ_m = jnp.where(is_src, occ, -1.0)
    n_slots = jnp.max(occ_m).astype(jnp.int32) + 1

    acc[0:half, :] = to

    def body(t, carry):
        pt = (n2t & (occ == t.astype(jnp.float32)) & is_src).astype(jnp.float32)
        contrib = jax.lax.dot_general(
            pt, te, (((0,), (0,)), ((), ())), precision=_HIGHEST
        )
        acc[0:half, :] = acc[0:half, :] + contrib
        return carry

    jax.lax.fori_loop(0, n_slots, body, 0)
    merged = acc[0:half, :]
    if r < half:
        kk = jax.lax.broadcasted_iota(jnp.int32, (half - r, half), 0) + r
        u = (rank_t == kk.astype(jnp.float32)).astype(jnp.float32)
        unm = jax.lax.dot(u, te, precision=_HIGHEST)
        merged = jnp.concatenate([unm, merged], axis=0)
    ss = merged[:, _C:]
    newx = merged[:, :_C] / _tile8(ss)
    return jnp.concatenate([newx, ss], axis=1)


def _fused_kernel(x_ref, w1_ref, b1_ref, w2_ref, b2_ref, o_ref, acc):
    x = x_ref[0]
    D = jnp.concatenate([x, jnp.ones((_T, _SZ), jnp.float32)], axis=1)
    D = _merge_step(D, acc, 288, 288)
    D = _merge_step(D, acc, 144, 144)
    D = _merge_step(D, acc, 72, 72)
    D = _merge_step(D, acc, 36, 8)
    y = D[:, :_C]
    h = jax.lax.dot(y, w1_ref[...]) + b1_ref[...]
    h = 0.5 * h * (1.0 + jax.lax.erf(h * np.float32(1.0 / np.sqrt(2.0))))
    out = jax.lax.dot(h, w2_ref[...]) + b2_ref[...]
    o_ref[0] = out


@jax.jit
def kernel(x, W1, b1, W2, b2):
    B, T, C = x.shape
    return pl.pallas_call(
        _fused_kernel,
        grid=(B,),
        in_specs=[
            pl.BlockSpec((1, T, C), lambda i: (i, 0, 0)),
            pl.BlockSpec((C, C), lambda i: (0, 0)),
            pl.BlockSpec((1, C), lambda i: (0, 0)),
            pl.BlockSpec((C, C), lambda i: (0, 0)),
            pl.BlockSpec((1, C), lambda i: (0, 0)),
        ],
        out_specs=pl.BlockSpec((1, 64, C), lambda i: (i, 0, 0)),
        out_shape=jax.ShapeDtypeStruct((B, 64, C), x.dtype),
        scratch_shapes=[pltpu.VMEM((288, _CA), jnp.float32)],
        compiler_params=pltpu.CompilerParams(
            dimension_semantics=("arbitrary",)
        ),
    )(x, W1, b1.reshape(1, C), W2, b2.reshape(1, C))
